# baseline (speedup 1.0000x reference)
import jax
import jax.numpy as jnp
from jax.experimental import pallas as pl
from jax.experimental.pallas import tpu as pltpu

_XHEAD = 16
_ZHEAD = 8
_XCHUNKS = 8
_ZCHUNKS = 2


def _fused_kernel(x_hbm, y_vmem, z_hbm, w_smem, xo_hbm, zo_hbm,
                  xh_vmem, zh_vmem, sem_big, sem_xh, sem_zh):
    nx = x_hbm.shape[0]
    nz = z_hbm.shape[0]
    xstep = ((nx - _XHEAD) // _XCHUNKS) // 8 * 8
    copies = []
    start = _XHEAD
    for k in range(_XCHUNKS):
        size = xstep if k < _XCHUNKS - 1 else nx - start
        c = pltpu.make_async_copy(
            x_hbm.at[pl.ds(start, size), :], xo_hbm.at[pl.ds(start, size), :],
            sem_big.at[k])
        c.start()
        copies.append(c)
        start += size
    zstep = ((nz - _ZHEAD) // _ZCHUNKS) // 8 * 8
    start = _ZHEAD
    for k in range(_ZCHUNKS):
        size = zstep if k < _ZCHUNKS - 1 else nz - start
        c = pltpu.make_async_copy(
            z_hbm.at[pl.ds(start, size), :], zo_hbm.at[pl.ds(start, size), :],
            sem_big.at[_XCHUNKS + k])
        c.start()
        copies.append(c)
        start += size

    xh_in = pltpu.make_async_copy(x_hbm.at[pl.ds(0, _XHEAD), :], xh_vmem, sem_xh)
    zh_in = pltpu.make_async_copy(z_hbm.at[pl.ds(0, _ZHEAD), :], zh_vmem, sem_zh)
    xh_in.start()
    zh_in.start()

    xh_in.wait()
    blk = xh_vmem[...]
    rows = jax.lax.broadcasted_iota(jnp.int32, blk.shape, 0)
    blk = jnp.where(rows == 10, y_vmem[0:1, :], blk)
    blk = jnp.where(rows == 2, y_vmem[1:2, :], blk)
    blk = jnp.where(rows == 1, jnp.float32(45.0), blk)
    xh_vmem[...] = blk
    xh_out = pltpu.make_async_copy(xh_vmem, xo_hbm.at[pl.ds(0, _XHEAD), :], sem_xh)
    xh_out.start()

    zh_in.wait()
    blk = zh_vmem[...]
    rows = jax.lax.broadcasted_iota(jnp.int32, blk.shape, 0)
    cols = jax.lax.broadcasted_iota(jnp.int32, blk.shape, 1)
    upd = jnp.where((rows == 1) & (cols == 3), w_smem[0], 0.0)
    upd = jnp.where((rows == 0) & (cols == 2), w_smem[1], upd)
    upd = jnp.where((rows == 0) & (cols == 1), w_smem[2], upd)
    zh_vmem[...] = blk + upd
    zh_out = pltpu.make_async_copy(zh_vmem, zo_hbm.at[pl.ds(0, _ZHEAD), :], sem_zh)
    zh_out.start()

    for c in copies:
        c.wait()
    xh_out.wait()
    zh_out.wait()


def kernel(x, y, z, w):
    return pl.pallas_call(
        _fused_kernel,
        in_specs=[
            pl.BlockSpec(memory_space=pl.ANY),
            pl.BlockSpec(memory_space=pltpu.VMEM),
            pl.BlockSpec(memory_space=pl.ANY),
            pl.BlockSpec(memory_space=pltpu.SMEM),
        ],
        out_specs=[
            pl.BlockSpec(memory_space=pl.ANY),
            pl.BlockSpec(memory_space=pl.ANY),
        ],
        out_shape=[
            jax.ShapeDtypeStruct(x.shape, x.dtype),
            jax.ShapeDtypeStruct(z.shape, z.dtype),
        ],
        scratch_shapes=[
            pltpu.VMEM((_XHEAD, x.shape[1]), jnp.float32),
            pltpu.VMEM((_ZHEAD, z.shape[1]), jnp.float32),
            pltpu.SemaphoreType.DMA((_XCHUNKS + _ZCHUNKS,)),
            pltpu.SemaphoreType.DMA,
            pltpu.SemaphoreType.DMA,
        ],
    )(x, y, z, w)


# speedup vs baseline: 48.3298x; 48.3298x over previous
import jax
import jax.numpy as jnp
from jax.experimental import pallas as pl
from jax.experimental.pallas import tpu as pltpu

_GRID = 64
_XBLK = 262144 // _GRID
_ZBLK = 16384 // _GRID


def _fused_kernel(x_ref, y_ref, z_ref, w_ref, xo_ref, zo_ref):
    i = pl.program_id(0)

    @pl.when(i != 0)
    def _plain():
        xo_ref[...] = x_ref[...]
        zo_ref[...] = z_ref[...]

    @pl.when(i == 0)
    def _fixup():
        xb = x_ref[...]
        rows = jax.lax.broadcasted_iota(jnp.int32, xb.shape, 0)
        xb = jnp.where(rows == 10, y_ref[0:1, :], xb)
        xb = jnp.where(rows == 2, y_ref[1:2, :], xb)
        xb = jnp.where(rows == 1, jnp.float32(45.0), xb)
        xo_ref[...] = xb

        zb = z_ref[...]
        rows = jax.lax.broadcasted_iota(jnp.int32, zb.shape, 0)
        cols = jax.lax.broadcasted_iota(jnp.int32, zb.shape, 1)
        upd = jnp.where((rows == 1) & (cols == 3), w_ref[0], 0.0)
        upd = jnp.where((rows == 0) & (cols == 2), w_ref[1], upd)
        upd = jnp.where((rows == 0) & (cols == 1), w_ref[2], upd)
        zo_ref[...] = zb + upd


def kernel(x, y, z, w):
    return pl.pallas_call(
        _fused_kernel,
        grid=(_GRID,),
        in_specs=[
            pl.BlockSpec((_XBLK, x.shape[1]), lambda i: (i, 0)),
            pl.BlockSpec((2, x.shape[1]), lambda i: (0, 0)),
            pl.BlockSpec((_ZBLK, z.shape[1]), lambda i: (i, 0)),
            pl.BlockSpec(memory_space=pltpu.SMEM),
        ],
        out_specs=[
            pl.BlockSpec((_XBLK, x.shape[1]), lambda i: (i, 0)),
            pl.BlockSpec((_ZBLK, z.shape[1]), lambda i: (i, 0)),
        ],
        out_shape=[
            jax.ShapeDtypeStruct(x.shape, x.dtype),
            jax.ShapeDtypeStruct(z.shape, z.dtype),
        ],
        compiler_params=pltpu.CompilerParams(dimension_semantics=("parallel",)),
    )(x, y, z, w)


# speedup vs baseline: 48.6586x; 1.0068x over previous
import jax
import jax.numpy as jnp
from jax.experimental import pallas as pl
from jax.experimental.pallas import tpu as pltpu

_GRID = 32
_XBLK = 262144 // _GRID
_ZBLK = 16384 // _GRID


def _fused_kernel(x_ref, y_ref, z_ref, w_ref, xo_ref, zo_ref):
    i = pl.program_id(0)

    @pl.when(i != 0)
    def _plain():
        xo_ref[...] = x_ref[...]
        zo_ref[...] = z_ref[...]

    @pl.when(i == 0)
    def _fixup():
        xb = x_ref[...]
        rows = jax.lax.broadcasted_iota(jnp.int32, xb.shape, 0)
        xb = jnp.where(rows == 10, y_ref[0:1, :], xb)
        xb = jnp.where(rows == 2, y_ref[1:2, :], xb)
        xb = jnp.where(rows == 1, jnp.float32(45.0), xb)
        xo_ref[...] = xb

        zb = z_ref[...]
        rows = jax.lax.broadcasted_iota(jnp.int32, zb.shape, 0)
        cols = jax.lax.broadcasted_iota(jnp.int32, zb.shape, 1)
        upd = jnp.where((rows == 1) & (cols == 3), w_ref[0], 0.0)
        upd = jnp.where((rows == 0) & (cols == 2), w_ref[1], upd)
        upd = jnp.where((rows == 0) & (cols == 1), w_ref[2], upd)
        zo_ref[...] = zb + upd


def kernel(x, y, z, w):
    return pl.pallas_call(
        _fused_kernel,
        grid=(_GRID,),
        in_specs=[
            pl.BlockSpec((_XBLK, x.shape[1]), lambda i: (i, 0)),
            pl.BlockSpec((2, x.shape[1]), lambda i: (0, 0)),
            pl.BlockSpec((_ZBLK, z.shape[1]), lambda i: (i, 0)),
            pl.BlockSpec(memory_space=pltpu.SMEM),
        ],
        out_specs=[
            pl.BlockSpec((_XBLK, x.shape[1]), lambda i: (i, 0)),
            pl.BlockSpec((_ZBLK, z.shape[1]), lambda i: (i, 0)),
        ],
        out_shape=[
            jax.ShapeDtypeStruct(x.shape, x.dtype),
            jax.ShapeDtypeStruct(z.shape, z.dtype),
        ],
        compiler_params=pltpu.CompilerParams(dimension_semantics=("parallel",)),
    )(x, y, z, w)
